# baseline (speedup 1.0000x reference)
import jax
import jax.numpy as jnp
from jax import lax
from jax.experimental import pallas as pl
from jax.experimental.pallas import tpu as pltpu

N = 16
D = 32
H = 64
BB = 2048

_C00 = (((0,), (0,)), ((), ()))


def _body(locs_ref, states_ref, w1_ref, b1_ref, w4_ref, w5_ref, b4_ref,
          b5_ref, pol_ref, val_ref, a0_ref):
    ft = states_ref[...].T
    lti = locs_ref[...].T

    r = lax.broadcasted_iota(jnp.int32, (2 * N, 2 * N), 0)
    s = lax.broadcasted_iota(jnp.int32, (2 * N, 2 * N), 1)
    perm = (s == 2 * (r % N) + r // N).astype(jnp.float32)
    lt = jnp.dot(perm, lti, preferred_element_type=jnp.float32)
    lx = lt[:N]
    ly = lt[N:]

    degs = []
    for i in range(N):
        dx = lx[i:i + 1] - lx
        dy = ly[i:i + 1] - ly
        a0row = (jnp.sqrt(dx * dx + dy * dy) < 0.99999994).astype(jnp.float32)
        a0_ref[i] = a0row
        degs.append(jnp.sum(a0row, axis=0, keepdims=True))
    dinv = lax.rsqrt(jnp.concatenate(degs, axis=0))

    gs = [ft[j * D:(j + 1) * D] * dinv[j:j + 1] for j in range(N)]
    glx = lx * dinv
    gly = ly * dinv

    w1 = w1_ref[...]
    w1a15 = w1[:D] * (N - 1.0)
    w1b = w1[D:2 * D]
    w1cb = jnp.concatenate([w1[2 * D:], (N - 1.0) * b1_ref[...]], axis=0)
    w45b = jnp.concatenate([
        jnp.concatenate([w4_ref[...], w5_ref[...]], axis=1),
        jnp.concatenate([b4_ref[...], b5_ref[...]], axis=1),
    ], axis=0)
    ones = jnp.ones((1, BB), jnp.float32)

    rows_p = []
    rows_v = []
    for i in range(N):
        a0row = a0_ref[i]
        di = dinv[i:i + 1]
        agg = a0row[0:1] * gs[0]
        for j in range(1, N):
            agg = agg + a0row[j:j + 1] * gs[j]
        acc = di * (agg - gs[i])

        t = jnp.sum(a0row * dinv, axis=0, keepdims=True)
        rs = di * t - di * di
        sx = jnp.sum(a0row * glx, axis=0, keepdims=True)
        sy = jnp.sum(a0row * gly, axis=0, keepdims=True)
        dsx = rs * lx[i:i + 1] - di * (sx - glx[i:i + 1])
        dsy = rs * ly[i:i + 1] - di * (sy - gly[i:i + 1])

        x = lax.dot_general(w1a15, ft[i * D:(i + 1) * D], _C00,
                            preferred_element_type=jnp.float32)
        x = x + lax.dot_general(w1b, acc, _C00,
                                preferred_element_type=jnp.float32)
        dse = jnp.concatenate([dsx, dsy, ones], axis=0)
        x = x + lax.dot_general(w1cb, dse, _C00,
                                preferred_element_type=jnp.float32)
        s2 = jnp.maximum(x, 0.0)
        s2e = jnp.concatenate([s2, ones], axis=0)
        pv = lax.dot_general(w45b, s2e, _C00,
                             preferred_element_type=jnp.float32)
        rows_p.append(pv[0:2])
        rows_v.append(pv[2:3])

    pol_ref[...] = jnp.concatenate(rows_p, axis=0).T
    val_ref[...] = jnp.concatenate(rows_v, axis=0).T


@jax.jit
def kernel(locs, states, W1, b1, W4, b4, W5, b5):
    B = locs.shape[0]
    G = B // BB

    pol, val = pl.pallas_call(
        _body,
        grid=(G,),
        in_specs=[
            pl.BlockSpec((BB, 2 * N), lambda g: (g, 0)),
            pl.BlockSpec((BB, N * D), lambda g: (g, 0)),
            pl.BlockSpec((2 * D + 2, H), lambda g: (0, 0)),
            pl.BlockSpec((1, H), lambda g: (0, 0)),
            pl.BlockSpec((H, 2), lambda g: (0, 0)),
            pl.BlockSpec((H, 1), lambda g: (0, 0)),
            pl.BlockSpec((1, 2), lambda g: (0, 0)),
            pl.BlockSpec((1, 1), lambda g: (0, 0)),
        ],
        out_specs=[
            pl.BlockSpec((BB, N * 2), lambda g: (g, 0)),
            pl.BlockSpec((BB, N), lambda g: (g, 0)),
        ],
        out_shape=[
            jax.ShapeDtypeStruct((B, N * 2), jnp.float32),
            jax.ShapeDtypeStruct((B, N), jnp.float32),
        ],
        scratch_shapes=[pltpu.VMEM((N, N, BB), jnp.float32)],
    )(locs.reshape(B, 2 * N), states.reshape(B, N * D), W1, b1[None, :],
      W4, W5, b4[None, :], b5[None, :])

    return pol.reshape(B, N, 2), val.reshape(B, N, 1)


# speedup vs baseline: 1.0107x; 1.0107x over previous
import jax
import jax.numpy as jnp
from jax import lax
from jax.experimental import pallas as pl
from jax.experimental.pallas import tpu as pltpu

N = 16
D = 32
H = 64
BB = 2048

_C00 = (((0,), (0,)), ((), ()))


def _body(locs_ref, states_ref, w1_ref, b1_ref, w4_ref, w5_ref, b4_ref,
          b5_ref, pol_ref, val_ref, a0_ref):
    ft = states_ref[...].T
    lti = locs_ref[...].T

    r = lax.broadcasted_iota(jnp.int32, (2 * N, 2 * N), 0)
    s = lax.broadcasted_iota(jnp.int32, (2 * N, 2 * N), 1)
    perm = (s == 2 * (r % N) + r // N).astype(jnp.float32)
    lt = jnp.dot(perm, lti, preferred_element_type=jnp.float32)
    lx = lt[:N]
    ly = lt[N:]

    degs = []
    for i in range(N):
        dx = lx[i:i + 1] - lx
        dy = ly[i:i + 1] - ly
        a0row = ((dx * dx + dy * dy) < 0.9999999).astype(jnp.float32)
        a0_ref[i] = a0row
        degs.append(jnp.sum(a0row, axis=0, keepdims=True))
    dinv = lax.rsqrt(jnp.concatenate(degs, axis=0))

    gs = [ft[j * D:(j + 1) * D] * dinv[j:j + 1] for j in range(N)]
    glx = lx * dinv
    gly = ly * dinv

    w1 = w1_ref[...]
    w1a15 = w1[:D] * (N - 1.0)
    w1b = w1[D:2 * D]
    w1cb = jnp.concatenate([w1[2 * D:], (N - 1.0) * b1_ref[...]], axis=0)
    w45b = jnp.concatenate([
        jnp.concatenate([w4_ref[...], w5_ref[...]], axis=1),
        jnp.concatenate([b4_ref[...], b5_ref[...]], axis=1),
    ], axis=0)
    ones = jnp.ones((1, BB), jnp.float32)

    rows_p = []
    rows_v = []
    for i in range(N):
        a0row = a0_ref[i]
        di = dinv[i:i + 1]
        agg = a0row[0:1] * gs[0]
        for j in range(1, N):
            agg = agg + a0row[j:j + 1] * gs[j]
        acc = di * (agg - gs[i])

        t = jnp.sum(a0row * dinv, axis=0, keepdims=True)
        rs = di * t - di * di
        sx = jnp.sum(a0row * glx, axis=0, keepdims=True)
        sy = jnp.sum(a0row * gly, axis=0, keepdims=True)
        dsx = rs * lx[i:i + 1] - di * (sx - glx[i:i + 1])
        dsy = rs * ly[i:i + 1] - di * (sy - gly[i:i + 1])

        x = lax.dot_general(w1a15, ft[i * D:(i + 1) * D], _C00,
                            preferred_element_type=jnp.float32)
        x = x + lax.dot_general(w1b, acc, _C00,
                                preferred_element_type=jnp.float32)
        dse = jnp.concatenate([dsx, dsy, ones], axis=0)
        x = x + lax.dot_general(w1cb, dse, _C00,
                                preferred_element_type=jnp.float32)
        s2 = jnp.maximum(x, 0.0)
        s2e = jnp.concatenate([s2, ones], axis=0)
        pv = lax.dot_general(w45b, s2e, _C00,
                             preferred_element_type=jnp.float32)
        rows_p.append(pv[0:2])
        rows_v.append(pv[2:3])

    pol_ref[...] = jnp.concatenate(rows_p, axis=0).T
    val_ref[...] = jnp.concatenate(rows_v, axis=0).T


@jax.jit
def kernel(locs, states, W1, b1, W4, b4, W5, b5):
    B = locs.shape[0]
    G = B // BB

    pol, val = pl.pallas_call(
        _body,
        grid=(G,),
        in_specs=[
            pl.BlockSpec((BB, 2 * N), lambda g: (g, 0)),
            pl.BlockSpec((BB, N * D), lambda g: (g, 0)),
            pl.BlockSpec((2 * D + 2, H), lambda g: (0, 0)),
            pl.BlockSpec((1, H), lambda g: (0, 0)),
            pl.BlockSpec((H, 2), lambda g: (0, 0)),
            pl.BlockSpec((H, 1), lambda g: (0, 0)),
            pl.BlockSpec((1, 2), lambda g: (0, 0)),
            pl.BlockSpec((1, 1), lambda g: (0, 0)),
        ],
        out_specs=[
            pl.BlockSpec((BB, N * 2), lambda g: (g, 0)),
            pl.BlockSpec((BB, N), lambda g: (g, 0)),
        ],
        out_shape=[
            jax.ShapeDtypeStruct((B, N * 2), jnp.float32),
            jax.ShapeDtypeStruct((B, N), jnp.float32),
        ],
        scratch_shapes=[pltpu.VMEM((N, N, BB), jnp.float32)],
    )(locs.reshape(B, 2 * N), states.reshape(B, N * D), W1, b1[None, :],
      W4, W5, b4[None, :], b5[None, :])

    return pol.reshape(B, N, 2), val.reshape(B, N, 1)
